# Initial kernel scaffold; baseline (speedup 1.0000x reference)
import jax
import jax.numpy as jnp
from jax.experimental import pallas as pl


def kernel(idx, token_embedding_table):
    raise NotImplementedError("write your pallas kernel here")


# speedup vs baseline: 1.0334x; 1.0334x over previous
import functools
import jax
import jax.numpy as jnp
from jax import lax
from jax.experimental import pallas as pl
from jax.experimental.pallas import tpu as pltpu
from jax.experimental.pallas import tpu_sc as plsc

VOCAB = 1000
BATCH = 1024
SEQ = 50
TOTAL = BATCH * SEQ
NUM_CORES = 2
NUM_SUBCORES = 16
NW = NUM_CORES * NUM_SUBCORES
BPW = TOTAL // NW
CHUNK = 40
NCHUNK = BPW // CHUNK

_mesh = plsc.VectorSubcoreMesh(core_axis_name="c", subcore_axis_name="s")


@functools.partial(
    pl.kernel,
    mesh=_mesh,
    out_type=jax.ShapeDtypeStruct((TOTAL, VOCAB), jnp.float32),
    scratch_types=[
        pltpu.VMEM((2, CHUNK), jnp.int32),
        pltpu.VMEM((2, CHUNK, VOCAB), jnp.float32),
        pltpu.SemaphoreType.DMA,
        pltpu.SemaphoreType.DMA,
    ],
    compiler_params=pltpu.CompilerParams(use_tc_tiling_on_sc=False),
)
def _gather_rows(table_hbm, idx_hbm, out_hbm, idx_v, rows_v, gsem, osem):
    wid = lax.axis_index("s") * NUM_CORES + lax.axis_index("c")
    base = wid * BPW

    def start_gather(j, slot):
        off = base + j * CHUNK
        pltpu.sync_copy(idx_hbm.at[pl.ds(off, CHUNK)], idx_v.at[slot])
        pltpu.async_copy(table_hbm.at[idx_v.at[slot]], rows_v.at[slot], gsem)

    start_gather(0, 0)

    def body(j, carry):
        slot = lax.rem(j, 2)
        nxt = 1 - slot
        off = base + j * CHUNK

        @pl.when(j > 0)
        def _():
            pltpu.make_async_copy(rows_v.at[nxt],
                                  out_hbm.at[pl.ds(off, CHUNK)], osem).wait()

        @pl.when(j + 1 < NCHUNK)
        def _():
            start_gather(j + 1, nxt)

        pltpu.make_async_copy(table_hbm.at[idx_v.at[slot]], rows_v.at[slot],
                              gsem).wait()
        pltpu.make_async_copy(rows_v.at[slot],
                              out_hbm.at[pl.ds(off, CHUNK)], osem).start()
        return carry

    lax.fori_loop(0, NCHUNK, body, 0)
    last_slot = (NCHUNK - 1) % 2
    off = base + (NCHUNK - 1) * CHUNK
    pltpu.make_async_copy(rows_v.at[last_slot],
                          out_hbm.at[pl.ds(off, CHUNK)], osem).wait()


def kernel(idx, token_embedding_table):
    flat = idx.reshape(-1).astype(jnp.int32)
    out = _gather_rows(token_embedding_table, flat)
    return out.reshape(idx.shape + (VOCAB,))
